# baseline (speedup 1.0000x reference)
import functools

import jax
import jax.numpy as jnp
from jax import lax
from jax.experimental import pallas as pl
from jax.experimental.pallas import tpu as pltpu
from jax.experimental.pallas import tpu_sc as plsc

_EMBED = 1024
_SEQ = 8192
_BATCH = 4
_NW = 32
_S_PER_W = _SEQ // _NW
_C = 16
_NCH = _S_PER_W // _C
_LANES = 16
_NBUF = 3
_R_BLK = 2048


def _sc_body(scale_hbm, x_hbm, pos_hbm, out_hbm, sbuf,
             pb0, pb1, xb0, xb1, xb2,
             spos0, spos1, sin0, sin1, sin2, sout0, sout1, sout2):
    wid = lax.axis_index("s") * 2 + lax.axis_index("c")
    pltpu.sync_copy(scale_hbm, sbuf)
    sv = sbuf[...]
    s_base = wid * _S_PER_W

    pbufs, sposs = (pb0, pb1), (spos0, spos1)
    xbufs = (xb0, xb1, xb2)
    sins = (sin0, sin1, sin2)
    souts = (sout0, sout1, sout2)

    def pos_cp(i):
        return pltpu.make_async_copy(
            pos_hbm.at[pl.ds(s_base + i * _C, _C), :], pbufs[i % 2], sposs[i % 2])

    def in_cp(t):
        return pltpu.make_async_copy(
            x_hbm.at[pl.ds(s_base + t * _C, _C), :], xbufs[t % _NBUF],
            sins[t % _NBUF])

    def out_cp(t):
        return pltpu.make_async_copy(
            xbufs[t % _NBUF], out_hbm.at[pl.ds(s_base + t * _C, _C), :],
            souts[t % _NBUF])

    def add_chunk(xbuf, pbuf):
        def body(k, carry):
            r = k // 8
            cc = (k % 8) * 128
            for u in range(8):
                sl = (r, pl.ds(cc + u * _LANES, _LANES))
                plsc.addupdate(xbuf.at[sl], pbuf[sl] * sv)
            return carry
        lax.fori_loop(0, _C * 8, body, 0)

    pos_cp(0).start()
    pos_cp(1).start()
    in_cp(0).start()
    in_cp(1).start()

    for t in range(_NCH):
        pos_cp(t).wait()
        in_cp(t).wait()
        add_chunk(xbufs[t % _NBUF], pbufs[t % 2])
        out_cp(t).start()
        if t + 2 < _NCH:
            if t >= 1:
                out_cp(t - 1).wait()
            pos_cp(t + 2).start()
            in_cp(t + 2).start()

    for t in range(_NCH - _NBUF, _NCH):
        out_cp(t).wait()


def _tc_body(scale_ref, x_ref, pos_ref, o_ref):
    o_ref[...] = x_ref[...] + scale_ref[0] * pos_ref[...]


def kernel(x, pos_table, use_pos_embed):
    batch, seq_len, embed_dim = x.shape
    scale = jnp.asarray(use_pos_embed, jnp.float32).reshape((1,))
    scale16 = jnp.full((_LANES,), scale[0])
    pos = pos_table[:seq_len]

    mesh = plsc.VectorSubcoreMesh(core_axis_name="c", subcore_axis_name="s")
    sc = functools.partial(
        pl.kernel,
        mesh=mesh,
        out_type=jax.ShapeDtypeStruct((seq_len, embed_dim), x.dtype),
        scratch_types=[
            pltpu.VMEM((_LANES,), jnp.float32),
            pltpu.VMEM((_C, _EMBED), jnp.float32),
            pltpu.VMEM((_C, _EMBED), jnp.float32),
            pltpu.VMEM((_C, _EMBED), jnp.float32),
            pltpu.VMEM((_C, _EMBED), jnp.float32),
            pltpu.VMEM((_C, _EMBED), jnp.float32),
            pltpu.SemaphoreType.DMA,
            pltpu.SemaphoreType.DMA,
            pltpu.SemaphoreType.DMA,
            pltpu.SemaphoreType.DMA,
            pltpu.SemaphoreType.DMA,
            pltpu.SemaphoreType.DMA,
            pltpu.SemaphoreType.DMA,
            pltpu.SemaphoreType.DMA,
        ],
        compiler_params=pltpu.CompilerParams(use_tc_tiling_on_sc=True),
    )(_sc_body)
    sc_out = sc(scale16, x[0], pos)

    x2d = x.reshape(batch * seq_len, embed_dim)
    period = seq_len // _R_BLK
    nb = batch - 1
    tc_out = pl.pallas_call(
        _tc_body,
        grid=(nb * period,),
        in_specs=[
            pl.BlockSpec(memory_space=pltpu.SMEM),
            pl.BlockSpec(
                (_R_BLK, embed_dim),
                lambda i: ((lax.rem(i, nb) + 1) * period + lax.div(i, nb), 0),
            ),
            pl.BlockSpec((_R_BLK, embed_dim), lambda i: (lax.div(i, nb), 0)),
        ],
        out_specs=pl.BlockSpec(
            (_R_BLK, embed_dim),
            lambda i: ((lax.rem(i, nb) + 1) * period + lax.div(i, nb), 0),
        ),
        out_shape=jax.ShapeDtypeStruct(x2d.shape, x.dtype),
        compiler_params=pltpu.CompilerParams(
            dimension_semantics=("arbitrary",),
        ),
    )(scale, x2d, pos)

    out = lax.dynamic_update_slice(tc_out, sc_out, (0, 0))
    return out.reshape(x.shape)


# speedup vs baseline: 1.2123x; 1.2123x over previous
import functools

import jax
import jax.numpy as jnp
from jax import lax
from jax.experimental import pallas as pl
from jax.experimental.pallas import tpu as pltpu
from jax.experimental.pallas import tpu_sc as plsc

_EMBED = 1024
_SEQ = 8192
_BATCH = 4
_NW = 32
_S_PER_W = _SEQ // _NW
_C = 16
_NCH = _S_PER_W // _C
_LANES = 16
_NBUF = 3


def _sc_body(scale_hbm, x_hbm, pos_hbm, out_hbm, sbuf,
             pb0, pb1, xb0, xb1, xb2,
             spos0, spos1, sin0, sin1, sin2, sout0, sout1, sout2):
    wid = lax.axis_index("s") * 2 + lax.axis_index("c")
    pltpu.sync_copy(scale_hbm, sbuf)
    sv = sbuf[...]
    s_base = wid * _S_PER_W

    pbufs, sposs = (pb0, pb1), (spos0, spos1)
    xbufs = (xb0, xb1, xb2)
    sins = (sin0, sin1, sin2)
    souts = (sout0, sout1, sout2)

    def pos_cp(i):
        return pltpu.make_async_copy(
            pos_hbm.at[pl.ds(s_base + i * _C, _C), :], pbufs[i % 2], sposs[i % 2])

    def in_cp(t):
        i, b = steps[t]
        return pltpu.make_async_copy(
            x_hbm.at[b, pl.ds(s_base + i * _C, _C), :], xbufs[t % _NBUF],
            sins[t % _NBUF])

    def out_cp(t):
        i, b = steps[t]
        return pltpu.make_async_copy(
            xbufs[t % _NBUF], out_hbm.at[b, pl.ds(s_base + i * _C, _C), :],
            souts[t % _NBUF])

    def add_chunk(xbuf, pbuf):
        def body(k, carry):
            r = k // 8
            cc = (k % 8) * 128
            for u in range(8):
                sl = (r, pl.ds(cc + u * _LANES, _LANES))
                xbuf[sl] = xbuf[sl] + pbuf[sl] * sv
            return carry
        lax.fori_loop(0, _C * 8, body, 0)

    steps = [(i, b) for i in range(_NCH) for b in range(_BATCH)]
    n = len(steps)

    pos_cp(0).start()
    pos_cp(1).start()
    in_cp(0).start()
    in_cp(1).start()

    for t in range(n):
        i, b = steps[t]
        if b == 0:
            pos_cp(i).wait()
        in_cp(t).wait()
        add_chunk(xbufs[t % _NBUF], pbufs[i % 2])
        out_cp(t).start()
        if b == _BATCH - 1 and i + 2 < _NCH:
            pos_cp(i + 2).start()
        if t + 2 < n:
            if t >= 1:
                out_cp(t - 1).wait()
            in_cp(t + 2).start()

    for t in range(n - _NBUF, n):
        out_cp(t).wait()


def kernel(x, pos_table, use_pos_embed):
    batch, seq_len, embed_dim = x.shape
    scale16 = jnp.full((_LANES,), jnp.asarray(use_pos_embed, jnp.float32))

    mesh = plsc.VectorSubcoreMesh(core_axis_name="c", subcore_axis_name="s")
    k = functools.partial(
        pl.kernel,
        mesh=mesh,
        out_type=jax.ShapeDtypeStruct(x.shape, x.dtype),
        scratch_types=[
            pltpu.VMEM((_LANES,), jnp.float32),
            pltpu.VMEM((_C, _EMBED), jnp.float32),
            pltpu.VMEM((_C, _EMBED), jnp.float32),
            pltpu.VMEM((_C, _EMBED), jnp.float32),
            pltpu.VMEM((_C, _EMBED), jnp.float32),
            pltpu.VMEM((_C, _EMBED), jnp.float32),
            pltpu.SemaphoreType.DMA,
            pltpu.SemaphoreType.DMA,
            pltpu.SemaphoreType.DMA,
            pltpu.SemaphoreType.DMA,
            pltpu.SemaphoreType.DMA,
            pltpu.SemaphoreType.DMA,
            pltpu.SemaphoreType.DMA,
            pltpu.SemaphoreType.DMA,
        ],
        compiler_params=pltpu.CompilerParams(use_tc_tiling_on_sc=True),
    )(_sc_body)
    return k(scale16, x, pos_table[:seq_len])
